# baseline (speedup 1.0000x reference)
import functools

import jax
import jax.numpy as jnp
from jax.experimental import pallas as pl

_EPS_BN = 1e-5
_TILE = 2048


def _norm_relu(y, st, lo, g, be, inv_n):
    m = st[lo : lo + 1, :] * inv_n
    v = st[lo + 1 : lo + 2, :] * inv_n - m * m
    inv = jax.lax.rsqrt(v + _EPS_BN)
    return jnp.maximum(g * (y - m) * inv + be, 0.0)


def _acc_moments(st_ref, y0, y2):
    st_ref[0:1, :] += jnp.sum(y0, axis=0, keepdims=True)
    st_ref[1:2, :] += jnp.sum(y0 * y0, axis=0, keepdims=True)
    st_ref[2:3, :] += jnp.sum(y2, axis=0, keepdims=True)
    st_ref[3:4, :] += jnp.sum(y2 * y2, axis=0, keepdims=True)


def _first_body(x0_ref, x2_ref, w_ref, b_ref, y0_ref, y2_ref, st_ref):
    i = pl.program_id(0)
    w = w_ref[...]
    b = b_ref[...]
    y0 = jnp.dot(x0_ref[...], w, preferred_element_type=jnp.float32) + b
    y2 = jnp.dot(x2_ref[...], w, preferred_element_type=jnp.float32) + b
    y0_ref[...] = y0
    y2_ref[...] = y2

    @pl.when(i == 0)
    def _():
        st_ref[...] = jnp.zeros(st_ref.shape, st_ref.dtype)

    _acc_moments(st_ref, y0, y2)


def _mid_body(inv_n, with_trl, *refs):
    if with_trl:
        (yp0_ref, yp2_ref, sin_ref, g_ref, be_ref, w_ref, b_ref, x0_ref,
         x2_ref, y0_ref, y2_ref, st_ref, trl_ref) = refs
    else:
        (yp0_ref, yp2_ref, sin_ref, g_ref, be_ref, w_ref, b_ref,
         y0_ref, y2_ref, st_ref) = refs
    i = pl.program_id(0)
    sin = sin_ref[...]
    g = g_ref[...]
    be = be_ref[...]
    h0 = _norm_relu(yp0_ref[...], sin, 0, g, be, inv_n)
    h2 = _norm_relu(yp2_ref[...], sin, 2, g, be, inv_n)
    if with_trl:
        d0 = x0_ref[...] - h0
        d2 = x2_ref[...] - h2

        @pl.when(i == 0)
        def _():
            trl_ref[...] = jnp.zeros(trl_ref.shape, trl_ref.dtype)

        trl_ref[0:1, :] += jnp.sum(d0 * d0, axis=0, keepdims=True)
        trl_ref[1:2, :] += jnp.sum(d2 * d2, axis=0, keepdims=True)
    w = w_ref[...]
    b = b_ref[...]
    y0 = jnp.dot(h0, w, preferred_element_type=jnp.float32) + b
    y2 = jnp.dot(h2, w, preferred_element_type=jnp.float32) + b
    y0_ref[...] = y0
    y2_ref[...] = y2

    @pl.when(i == 0)
    def _():
        st_ref[...] = jnp.zeros(st_ref.shape, st_ref.dtype)

    _acc_moments(st_ref, y0, y2)


def _pred_body(inv_n, yp0_ref, yp2_ref, sin_ref, g_ref, be_ref, origin_ref,
               we_ref, p0_ref, p2_ref, st_ref):
    i = pl.program_id(0)
    sin = sin_ref[...]
    g = g_ref[...]
    be = be_ref[...]
    p0 = _norm_relu(yp0_ref[...], sin, 0, g, be, inv_n)
    p2 = _norm_relu(yp2_ref[...], sin, 2, g, be, inv_n)
    p0_ref[...] = p0
    p2_ref[...] = p2
    o = origin_ref[...]
    we = we_ref[...]
    wo = we[:, 3:4] != 0
    m0 = (we[:, 0:1] != 0) & wo
    m2 = (we[:, 2:3] != 0) & wo
    na = jnp.maximum(jnp.abs(o), 1e-8)
    r0 = (o * p0) / (na * jnp.maximum(p0, 1e-8))
    r2 = (o * p2) / (na * jnp.maximum(p2, 1e-8))

    @pl.when(i == 0)
    def _():
        st_ref[...] = jnp.zeros(st_ref.shape, st_ref.dtype)

    st_ref[0:1, :] += jnp.sum(jnp.where(m0, r0, 0.0), axis=0, keepdims=True)
    st_ref[1:2, :] = st_ref[1:2, :] + jnp.sum(m0.astype(jnp.float32))
    st_ref[2:3, :] += jnp.sum(jnp.where(m2, r2, 0.0), axis=0, keepdims=True)
    st_ref[3:4, :] = st_ref[3:4, :] + jnp.sum(m2.astype(jnp.float32))


def _combine_body(inv_nd, p0_ref, p2_ref, origin_ref, we_ref, st_ref, trl_ref,
                  out_ref, l1_ref, l2_ref):
    i = pl.program_id(0)
    st = st_ref[...]
    r0 = jnp.sum(st[0:1, :])
    c0 = jnp.sum(st[1:2, :])
    r2 = jnp.sum(st[2:3, :])
    c2 = jnp.sum(st[3:4, :])
    tcl0 = jnp.where(c0 > 0.0, r0 / jnp.maximum(c0, 1.0), 0.0)
    tcl2 = jnp.where(c2 > 0.0, r2 / jnp.maximum(c2, 1.0), 0.0)
    sel2 = tcl2 >= tcl0
    we = we_ref[...]
    wo = we[:, 3:4] != 0
    m0 = (we[:, 0:1] != 0) & (~wo)
    m2 = (we[:, 2:3] != 0) & (~wo)
    p0 = p0_ref[...]
    p2 = p2_ref[...]
    best = jnp.where(sel2, p2, p0)
    pick = jnp.where(m0 & m2, best,
                     jnp.where(m0, p0, jnp.where(m2, p2, 0.0)))
    out_ref[...] = jnp.where(wo, origin_ref[...], pick) + 1e-6

    @pl.when(i == 0)
    def _():
        l1_ref[...] = jnp.reshape((tcl0 + tcl2) * 0.5, (1, 1))
        l2_ref[...] = jnp.reshape(jnp.sum(trl_ref[...]) * inv_nd, (1, 1))


def kernel(zs, we, origin_index, enc_W, enc_b, enc_g, enc_be,
           dec_W, dec_b, dec_g, dec_be):
    n, d = zs.shape[1], zs.shape[2]
    tile = min(_TILE, n)
    grid = (n // tile,)
    f32 = jnp.float32
    inv_n = 1.0 / float(n)

    x0 = zs[0]
    x2 = zs[2]
    origin = zs[origin_index]
    we4 = jnp.concatenate([we, we[:, origin_index][:, None]], axis=1)

    row_spec = pl.BlockSpec((tile, d), lambda i: (i, 0))
    we_spec = pl.BlockSpec((tile, 4), lambda i: (i, 0))
    st_spec = pl.BlockSpec((4, d), lambda i: (0, 0))
    trl_spec = pl.BlockSpec((2, d), lambda i: (0, 0))
    vec_spec = pl.BlockSpec((1, d), lambda i: (0, 0))
    w_spec = pl.BlockSpec((d, d), lambda i: (0, 0))
    one_spec = pl.BlockSpec((1, 1), lambda i: (0, 0))

    row_out = jax.ShapeDtypeStruct((n, d), f32)
    st_out = jax.ShapeDtypeStruct((4, d), f32)

    Ws = [enc_W[0], enc_W[1], enc_W[2], dec_W[0], dec_W[1]]
    bs = [enc_b[0], enc_b[1], enc_b[2], dec_b[0], dec_b[1]]
    gs = [enc_g[0], enc_g[1], enc_g[2], dec_g[0], dec_g[1]]
    bes = [enc_be[0], enc_be[1], enc_be[2], dec_be[0], dec_be[1]]
    bs = [b.reshape(1, d) for b in bs]
    gs = [g.reshape(1, d) for g in gs]
    bes = [b.reshape(1, d) for b in bes]

    y0, y2, st = pl.pallas_call(
        _first_body,
        grid=grid,
        in_specs=[row_spec, row_spec, w_spec, vec_spec],
        out_specs=[row_spec, row_spec, st_spec],
        out_shape=[row_out, row_out, st_out],
    )(x0, x2, Ws[0], bs[0])

    trl = None
    for k in (1, 2, 3, 4):
        with_trl = k == 3
        body = functools.partial(_mid_body, inv_n, with_trl)
        in_specs = [row_spec, row_spec, st_spec, vec_spec, vec_spec,
                    w_spec, vec_spec]
        args = [y0, y2, st, gs[k - 1], bes[k - 1], Ws[k], bs[k]]
        out_specs = [row_spec, row_spec, st_spec]
        out_shape = [row_out, row_out, st_out]
        if with_trl:
            in_specs += [row_spec, row_spec]
            args += [x0, x2]
            out_specs += [trl_spec]
            out_shape += [jax.ShapeDtypeStruct((2, d), f32)]
        res = pl.pallas_call(
            body,
            grid=grid,
            in_specs=in_specs,
            out_specs=out_specs,
            out_shape=out_shape,
        )(*args)
        if with_trl:
            y0, y2, st, trl = res
        else:
            y0, y2, st = res

    p0, p2, st6 = pl.pallas_call(
        functools.partial(_pred_body, inv_n),
        grid=grid,
        in_specs=[row_spec, row_spec, st_spec, vec_spec, vec_spec,
                  row_spec, we_spec],
        out_specs=[row_spec, row_spec, st_spec],
        out_shape=[row_out, row_out, st_out],
    )(y0, y2, st, gs[4], bes[4], origin, we4)

    comp, l1, l2 = pl.pallas_call(
        functools.partial(_combine_body, 1.0 / (float(n) * float(d))),
        grid=grid,
        in_specs=[row_spec, row_spec, row_spec, we_spec, st_spec, trl_spec],
        out_specs=[row_spec, one_spec, one_spec],
        out_shape=[row_out, jax.ShapeDtypeStruct((1, 1), f32),
                   jax.ShapeDtypeStruct((1, 1), f32)],
    )(p0, p2, origin, we4, st6, trl)

    return comp, l1.reshape(()), l2.reshape(())


# speedup vs baseline: 1.2765x; 1.2765x over previous
import functools

import jax
import jax.numpy as jnp
from jax import lax
from jax.experimental import pallas as pl
from jax.experimental.pallas import tpu as pltpu

_EPS_BN = 1e-5
_TP = 1024


def _halves(v):
    return v[:, 0:64], v[:, 64:128]


def _wide(a, b, t):
    ai = a.astype(jnp.int32)
    bi = b.astype(jnp.int32)
    w = jnp.concatenate([jnp.broadcast_to(ai, (t, 64)),
                         jnp.broadcast_to(bi, (t, 64))], axis=1)
    return w != 0


def _run_view(x_ref, w_ref, b_ref, g_ref, be_ref, h_ref, n, n2):
    f32 = jnp.float32
    nc = n2 // _TP
    inv_n = 1.0 / float(n)
    z = jnp.zeros((1, 128), f32)
    stats = None
    trl_cols = z
    for k in range(5):
        w = w_ref[k]
        b = b_ref[k : k + 1, :]
        if k > 0:
            g = g_ref[k - 1 : k, :]
            be = be_ref[k - 1 : k, :]
            m, inv = stats

        def body(c, carry, k=k):
            s, sq, trl = carry
            sl = pl.ds(c * _TP, _TP)
            if k == 0:
                hv = x_ref[sl, :]
            else:
                hv = jnp.maximum(g * (h_ref[sl, :] - m) * inv + be, 0.0)
            if k == 3:
                dd = x_ref[sl, :] - hv
                trl = trl + jnp.sum(dd * dd, axis=0, keepdims=True)
            y = jnp.dot(hv, w, preferred_element_type=f32) + b
            h_ref[sl, :] = y
            s = s + jnp.sum(y, axis=0, keepdims=True)
            sq = sq + jnp.sum(y * y, axis=0, keepdims=True)
            return s, sq, trl

        s, sq, trl_new = lax.fori_loop(0, nc, body, (z, z, z))
        sa, sb = _halves(s)
        qa, qb = _halves(sq)
        m64 = (sa + sb) * inv_n
        v64 = (qa + qb) * inv_n - m64 * m64
        i64 = jax.lax.rsqrt(v64 + _EPS_BN)
        stats = (jnp.concatenate([m64, m64], axis=1),
                 jnp.concatenate([i64, i64], axis=1))
        if k == 3:
            trl_cols = trl_new

    g = g_ref[4:5, :]
    be = be_ref[4:5, :]
    m, inv = stats

    def pbody(c, _):
        sl = pl.ds(c * _TP, _TP)
        h_ref[sl, :] = jnp.maximum(g * (h_ref[sl, :] - m) * inv + be, 0.0)
        return 0

    lax.fori_loop(0, nc, pbody, 0)
    return trl_cols


def _ratio(o_t, pred_t, mk):
    na = jnp.maximum(jnp.abs(o_t), 1e-8)
    r = (o_t * pred_t) / (na * jnp.maximum(pred_t, 1e-8))
    return (jnp.sum(jnp.where(mk, r, 0.0), axis=0, keepdims=True),
            jnp.sum(mk.astype(jnp.float32), axis=0, keepdims=True))


def _viewA_body(n, n2, x_ref, o_ref, cd_ref, w_ref, b_ref, g_ref, be_ref,
                t0_ref, st_ref, h_ref, acc_ref):
    i = pl.program_id(0)
    nt = n2 // _TP

    @pl.when(i == 0)
    def _():
        acc_ref[...] = jnp.zeros(acc_ref.shape, acc_ref.dtype)
        trl = _run_view(x_ref, w_ref, b_ref, g_ref, be_ref, h_ref, n, n2)
        acc_ref[2:3, :] = trl

    pred_t = h_ref[pl.ds(i * _TP, _TP), :]
    o_t = o_ref[...]
    cd = cd_ref[...]
    c0 = cd[:, 0:1]
    c1 = cd[:, 1:2]
    mk0 = _wide(((c0 & 1) != 0) & ((c0 & 4) != 0),
                ((c1 & 1) != 0) & ((c1 & 4) != 0), _TP)
    zero = _wide((c0 & 7) == 0, (c1 & 7) == 0, _TP)
    rs, cn = _ratio(o_t, pred_t, mk0)
    acc_ref[0:1, :] += rs
    acc_ref[1:2, :] += cn
    t0_ref[...] = jnp.where(zero, 1e-6, pred_t + 1e-6)

    @pl.when(i == nt - 1)
    def _():
        st_ref[...] = acc_ref[...]


def _viewB_body(n, n2, inv_nd, x_ref, o_ref, cd_ref, t0_ref, st0_ref, w_ref,
                b_ref, g_ref, be_ref, comp_ref, l1_ref, l2_ref, h_ref,
                acc_ref):
    q = pl.program_id(0)
    i = pl.program_id(1)
    nt = n2 // _TP

    @pl.when((q == 0) & (i == 0))
    def _():
        acc_ref[...] = jnp.zeros(acc_ref.shape, acc_ref.dtype)
        trl = _run_view(x_ref, w_ref, b_ref, g_ref, be_ref, h_ref, n, n2)
        acc_ref[2:3, :] = trl

    pred_t = h_ref[pl.ds(i * _TP, _TP), :]
    o_t = o_ref[...]
    cd = cd_ref[...]
    c0 = cd[:, 0:1]
    c1 = cd[:, 1:2]

    @pl.when(q == 0)
    def _():
        mk2 = _wide(((c0 & 2) != 0) & ((c0 & 4) != 0),
                    ((c1 & 2) != 0) & ((c1 & 4) != 0), _TP)
        rs, cn = _ratio(o_t, pred_t, mk2)
        acc_ref[0:1, :] += rs
        acc_ref[1:2, :] += cn

    @pl.when(q == 1)
    def _():
        st0 = st0_ref[...]
        acc = acc_ref[...]
        r0 = jnp.sum(st0[0:1, :])
        c0t = jnp.sum(st0[1:2, :])
        tcl0 = jnp.where(c0t > 0.0, r0 / jnp.maximum(c0t, 1.0), 0.0)
        r2 = jnp.sum(acc[0:1, :])
        c2t = jnp.sum(acc[1:2, :])
        tcl2 = jnp.where(c2t > 0.0, r2 / jnp.maximum(c2t, 1.0), 0.0)
        sel = tcl2 >= tcl0
        wo = _wide((c0 & 4) != 0, (c1 & 4) != 0, _TP)
        m0 = _wide(((c0 & 1) != 0) & ((c0 & 4) == 0),
                   ((c1 & 1) != 0) & ((c1 & 4) == 0), _TP)
        m2 = _wide(((c0 & 2) != 0) & ((c0 & 4) == 0),
                   ((c1 & 2) != 0) & ((c1 & 4) == 0), _TP)
        t0v = t0_ref[...]
        comp_ref[...] = jnp.where(
            wo, o_t + 1e-6,
            jnp.where(m2 & (sel | (~m0)), pred_t + 1e-6, t0v))

        @pl.when(i == nt - 1)
        def _():
            l1_ref[...] = jnp.reshape((tcl0 + tcl2) * 0.5, (1, 1))
            l2_ref[...] = jnp.reshape(
                (jnp.sum(st0[2:3, :]) + jnp.sum(acc[2:3, :])) * inv_nd,
                (1, 1))


def kernel(zs, we, origin_index, enc_W, enc_b, enc_g, enc_be,
           dec_W, dec_b, dec_g, dec_be):
    n, d = zs.shape[1], zs.shape[2]
    n2 = n // 2
    nt = n2 // _TP
    f32 = jnp.float32

    x0 = zs[0].reshape(n2, 128)
    x2 = zs[2].reshape(n2, 128)
    origin = zs[origin_index].reshape(n2, 128)
    wo_col = we[:, origin_index]
    code = ((we[:, 0] != 0).astype(jnp.int32)
            + 2 * (we[:, 2] != 0).astype(jnp.int32)
            + 4 * (wo_col != 0).astype(jnp.int32))
    codes2 = code.reshape(n2, 2)

    wst = jnp.concatenate([enc_W, dec_W], axis=0)
    zW = jnp.zeros_like(wst)
    w128 = jnp.concatenate([jnp.concatenate([wst, zW], axis=2),
                            jnp.concatenate([zW, wst], axis=2)], axis=1)
    b128 = jnp.tile(jnp.concatenate([enc_b, dec_b], axis=0), (1, 2))
    g128 = jnp.tile(jnp.concatenate([enc_g, dec_g], axis=0), (1, 2))
    be128 = jnp.tile(jnp.concatenate([enc_be, dec_be], axis=0), (1, 2))

    def fullA(shape):
        return pl.BlockSpec(shape, lambda i: tuple(0 for _ in shape))

    def fullB(shape):
        return pl.BlockSpec(shape, lambda q, i: tuple(0 for _ in shape))

    t0, st0 = pl.pallas_call(
        functools.partial(_viewA_body, n, n2),
        grid=(nt,),
        in_specs=[fullA((n2, 128)),
                  pl.BlockSpec((_TP, 128), lambda i: (i, 0)),
                  pl.BlockSpec((_TP, 2), lambda i: (i, 0)),
                  fullA((5, 128, 128)), fullA((5, 128)), fullA((5, 128)),
                  fullA((5, 128))],
        out_specs=[pl.BlockSpec((_TP, 128), lambda i: (i, 0)),
                   fullA((4, 128))],
        out_shape=[jax.ShapeDtypeStruct((n2, 128), f32),
                   jax.ShapeDtypeStruct((4, 128), f32)],
        scratch_shapes=[pltpu.VMEM((n2, 128), f32),
                        pltpu.VMEM((4, 128), f32)],
    )(x0, origin, codes2, w128, b128, g128, be128)

    comp, l1, l2 = pl.pallas_call(
        functools.partial(_viewB_body, n, n2, 1.0 / (float(n) * float(d))),
        grid=(2, nt),
        in_specs=[fullB((n2, 128)),
                  pl.BlockSpec((_TP, 128), lambda q, i: (i, 0)),
                  pl.BlockSpec((_TP, 2), lambda q, i: (i, 0)),
                  pl.BlockSpec((_TP, 128), lambda q, i: (i * q, 0)),
                  fullB((4, 128)),
                  fullB((5, 128, 128)), fullB((5, 128)), fullB((5, 128)),
                  fullB((5, 128))],
        out_specs=[pl.BlockSpec((_TP, 128), lambda q, i: (i * q, 0)),
                   fullB((1, 1)), fullB((1, 1))],
        out_shape=[jax.ShapeDtypeStruct((n2, 128), f32),
                   jax.ShapeDtypeStruct((1, 1), f32),
                   jax.ShapeDtypeStruct((1, 1), f32)],
        scratch_shapes=[pltpu.VMEM((n2, 128), f32),
                        pltpu.VMEM((4, 128), f32)],
    )(x2, origin, codes2, t0, st0, w128, b128, g128, be128)

    return comp.reshape(n, d), l1.reshape(()), l2.reshape(())
